# baseline (speedup 1.0000x reference)
import functools

import jax
import jax.numpy as jnp
from jax.experimental import pallas as pl
import jax.experimental.pallas.tpu as pltpu

_NBUF = 3


def _pick_block(n, target):
    best = None
    for d in range(8, min(n, target) + 1, 8):
        if n % d == 0:
            best = d
    if best is not None:
        return best
    for d in range(min(n, target), 0, -1):
        if n % d == 0:
            return d
    return n


def _gcn_kernel(x_hbm, w_ref, adj_hbm, out_ref, sup_ref, *bufs_sems,
                bm, xc, num_m, num_xc):
    adj_bufs = bufs_sems[:_NBUF]
    x_bufs = bufs_sems[_NBUF:_NBUF + 2]
    adj_sems = bufs_sems[_NBUF + 2:2 * _NBUF + 2]
    x_sems = bufs_sems[2 * _NBUF + 2:]

    m = pl.program_id(0)

    def adj_copy(t, b):
        return pltpu.make_async_copy(
            adj_hbm.at[pl.ds(t * bm, bm), :], adj_bufs[b], adj_sems[b]
        )

    def x_copy(c):
        return pltpu.make_async_copy(
            x_hbm.at[pl.ds(c * xc, xc), :], x_bufs[c % 2], x_sems[c % 2]
        )

    @pl.when(m == 0)
    def _prologue():
        for b in range(min(_NBUF, num_m)):
            adj_copy(b, b).start()
        for c in range(min(2, num_xc)):
            x_copy(c).start()
        for c in range(num_xc):
            x_copy(c).wait()
            sup_ref[pl.ds(c * xc, xc), :] = jnp.dot(
                x_bufs[c % 2][...], w_ref[...],
                preferred_element_type=jnp.float32,
            ).astype(sup_ref.dtype)
            if c + 2 < num_xc:
                x_copy(c + 2).start()

    slot = jax.lax.rem(m, _NBUF)
    for b in range(_NBUF):

        @pl.when(slot == b)
        def _step(b=b):
            adj_copy(m, b).wait()
            out_ref[...] = jnp.maximum(
                jnp.dot(
                    adj_bufs[b][...], sup_ref[...].astype(jnp.float32),
                    preferred_element_type=jnp.float32,
                ),
                0.0,
            )

            @pl.when(m + _NBUF < num_m)
            def _refill():
                adj_copy(m + _NBUF, b).start()


@jax.jit
def kernel(x_features, adj, weight):
    n, in_f = x_features.shape
    out_f = weight.shape[1]

    bm = _pick_block(n, 400)
    xc = _pick_block(n, 2000)
    num_m = n // bm
    num_xc = n // xc

    return pl.pallas_call(
        functools.partial(
            _gcn_kernel, bm=bm, xc=xc, num_m=num_m, num_xc=num_xc
        ),
        grid=(num_m,),
        in_specs=[
            pl.BlockSpec(memory_space=pl.ANY),
            pl.BlockSpec((in_f, out_f), lambda m: (0, 0)),
            pl.BlockSpec(memory_space=pl.ANY),
        ],
        out_specs=pl.BlockSpec((bm, out_f), lambda m: (m, 0)),
        out_shape=jax.ShapeDtypeStruct((n, out_f), jnp.float32),
        scratch_shapes=(
            [pltpu.VMEM((n, out_f), jnp.bfloat16)]
            + [pltpu.VMEM((bm, n), jnp.float32) for _ in range(_NBUF)]
            + [pltpu.VMEM((xc, in_f), jnp.float32) for _ in range(2)]
            + [pltpu.SemaphoreType.DMA for _ in range(_NBUF + 2)]
        ),
        compiler_params=pltpu.CompilerParams(
            vmem_limit_bytes=64 * 1024 * 1024
        ),
    )(x_features, weight, adj)


# speedup vs baseline: 1.0190x; 1.0190x over previous
import functools

import jax
import jax.numpy as jnp
from jax.experimental import pallas as pl
import jax.experimental.pallas.tpu as pltpu

_NBUF = 3


def _pick_block(n, target):
    best = None
    for d in range(8, min(n, target) + 1, 8):
        if n % d == 0:
            best = d
    if best is not None:
        return best
    for d in range(min(n, target), 0, -1):
        if n % d == 0:
            return d
    return n


def _gcn_kernel(x_hbm, w_ref, adj_hbm, out_ref, sup_ref, *bufs_sems,
                bm, xc, num_m, num_xc):
    adj_bufs = bufs_sems[:_NBUF]
    x_bufs = bufs_sems[_NBUF:_NBUF + 2]
    adj_sems = bufs_sems[_NBUF + 2:2 * _NBUF + 2]
    x_sems = bufs_sems[2 * _NBUF + 2:]

    m = pl.program_id(0)

    def adj_copy(t, b):
        return pltpu.make_async_copy(
            adj_hbm.at[pl.ds(t * bm, bm), :], adj_bufs[b], adj_sems[b]
        )

    def x_copy(c):
        return pltpu.make_async_copy(
            x_hbm.at[pl.ds(c * xc, xc), :], x_bufs[c % 2], x_sems[c % 2]
        )

    @pl.when(m == 0)
    def _prologue():
        for b in range(min(2, num_m)):
            adj_copy(b, b).start()
        for c in range(min(2, num_xc)):
            x_copy(c).start()
        for c in range(num_xc):
            x_copy(c).wait()
            sup_ref[pl.ds(c * xc, xc), :] = jnp.dot(
                x_bufs[c % 2][...], w_ref[...],
                preferred_element_type=jnp.float32,
            ).astype(sup_ref.dtype)
            if c + 2 < num_xc:
                x_copy(c + 2).start()

    nxt = m + 2
    nxt_slot = jax.lax.rem(nxt, _NBUF)
    for b in range(_NBUF):

        @pl.when((nxt_slot == b) & (nxt < num_m))
        def _refill(b=b):
            adj_copy(nxt, b).start()

    slot = jax.lax.rem(m, _NBUF)
    for b in range(_NBUF):

        @pl.when(slot == b)
        def _step(b=b):
            adj_copy(m, b).wait()
            out_ref[...] = jnp.maximum(
                jnp.dot(
                    adj_bufs[b][...], sup_ref[...].astype(jnp.float32),
                    preferred_element_type=jnp.float32,
                ),
                0.0,
            )


@jax.jit
def kernel(x_features, adj, weight):
    n, in_f = x_features.shape
    out_f = weight.shape[1]

    bm = _pick_block(n, 400)
    xc = _pick_block(n, 2000)
    num_m = n // bm
    num_xc = n // xc

    return pl.pallas_call(
        functools.partial(
            _gcn_kernel, bm=bm, xc=xc, num_m=num_m, num_xc=num_xc
        ),
        grid=(num_m,),
        in_specs=[
            pl.BlockSpec(memory_space=pl.ANY),
            pl.BlockSpec((in_f, out_f), lambda m: (0, 0)),
            pl.BlockSpec(memory_space=pl.ANY),
        ],
        out_specs=pl.BlockSpec((bm, out_f), lambda m: (m, 0)),
        out_shape=jax.ShapeDtypeStruct((n, out_f), jnp.float32),
        scratch_shapes=(
            [pltpu.VMEM((n, out_f), jnp.bfloat16)]
            + [pltpu.VMEM((bm, n), jnp.float32) for _ in range(_NBUF)]
            + [pltpu.VMEM((xc, in_f), jnp.float32) for _ in range(2)]
            + [pltpu.SemaphoreType.DMA for _ in range(_NBUF + 2)]
        ),
        compiler_params=pltpu.CompilerParams(
            vmem_limit_bytes=64 * 1024 * 1024
        ),
    )(x_features, weight, adj)
